# Initial kernel scaffold; baseline (speedup 1.0000x reference)
import functools

import jax
import jax.numpy as jnp
from jax import lax
from jax.experimental import pallas as pl
from jax.experimental.pallas import tpu as pltpu
from jax.experimental.pallas import tpu_sc as plsc

NC = 2
NS = 16
CH = 128
D = 128


def _sc_degree(dsts, zdeg, nch, npad, rps):
    mesh = plsc.VectorSubcoreMesh(core_axis_name="c", subcore_axis_name="s")

    @functools.partial(
        pl.kernel,
        out_type=jax.ShapeDtypeStruct((NC, npad), jnp.float32),
        mesh=mesh,
        scratch_types=[
            pltpu.VMEM((nch, CH), jnp.int32),
            pltpu.VMEM((CH,), jnp.float32),
            pltpu.VMEM_SHARED((npad,), jnp.float32),
        ],
    )
    def deg_kernel(dsts_hbm, zdeg_hbm, dp_hbm, idx_v, ones_v, sdeg):
        c = lax.axis_index("c")
        s = lax.axis_index("s")
        for i in range(CH // 16):
            ones_v[pl.ds(i * 16, 16)] = jnp.full((16,), 1.0, jnp.float32)
        pltpu.sync_copy(dsts_hbm.at[c, s], idx_v)
        pltpu.sync_copy(zdeg_hbm, sdeg.at[pl.ds(s * rps, rps)])
        plsc.subcore_barrier()

        def body(j, carry):
            pltpu.sync_copy(ones_v, sdeg.at[idx_v.at[j]], add=True)
            return carry

        lax.fori_loop(0, nch, body, 0)
        plsc.subcore_barrier()
        pltpu.sync_copy(sdeg.at[pl.ds(s * rps, rps)],
                        dp_hbm.at[c, pl.ds(s * rps, rps)])

    return deg_kernel(dsts, zdeg)


def _sc_aggregate(g, srcs, dsts, zrows, nch, npad, rps):
    mesh = plsc.VectorSubcoreMesh(core_axis_name="c", subcore_axis_name="s")

    @functools.partial(
        pl.kernel,
        out_type=jax.ShapeDtypeStruct((NC, npad, D), jnp.float32),
        mesh=mesh,
        scratch_types=[
            pltpu.VMEM((nch, CH), jnp.int32),
            pltpu.VMEM((nch, CH), jnp.int32),
            pltpu.VMEM((2, CH, D), jnp.float32),
            pltpu.VMEM_SHARED((npad, D), jnp.float32),
            pltpu.SemaphoreType.DMA,
            pltpu.SemaphoreType.DMA,
        ],
    )
    def agg_kernel(g_hbm, srcs_hbm, dsts_hbm, zrows_hbm, s_hbm,
                   isrc, idst, rows, acc, sem0, sem1):
        c = lax.axis_index("c")
        s = lax.axis_index("s")
        pltpu.sync_copy(srcs_hbm.at[c, s], isrc)
        pltpu.sync_copy(dsts_hbm.at[c, s], idst)
        pltpu.sync_copy(zrows_hbm, acc.at[pl.ds(s * rps, rps)])
        plsc.subcore_barrier()

        pltpu.async_copy(g_hbm.at[isrc.at[0]], rows.at[0], sem0)

        def body(k, carry):
            j = 2 * k
            pltpu.async_copy(g_hbm.at[isrc.at[j + 1]], rows.at[1], sem1)
            pltpu.make_async_copy(g_hbm.at[pl.ds(0, CH)], rows.at[0], sem0).wait()
            pltpu.sync_copy(rows.at[0], acc.at[idst.at[j]], add=True)

            @pl.when(k + 1 < nch // 2)
            def _():
                pltpu.async_copy(g_hbm.at[isrc.at[j + 2]], rows.at[0], sem0)

            pltpu.make_async_copy(g_hbm.at[pl.ds(0, CH)], rows.at[1], sem1).wait()
            pltpu.sync_copy(rows.at[1], acc.at[idst.at[j + 1]], add=True)
            return carry

        lax.fori_loop(0, nch // 2, body, 0)
        plsc.subcore_barrier()
        pltpu.sync_copy(acc.at[pl.ds(s * rps, rps)],
                        s_hbm.at[c, pl.ds(s * rps, rps)])

    return agg_kernel(g, srcs, dsts, zrows)


def _tc_first(dp0, dp1, x, w1, npad, blk):
    grid = (npad // blk,)

    def body(dp0_ref, dp1_ref, x_ref, w_ref, g_ref, dinv_ref):
        d = lax.rsqrt(dp0_ref[...] + dp1_ref[...] + 1.0)
        dinv_ref[...] = d
        g_ref[...] = d * jnp.dot(x_ref[...], w_ref[...],
                                 preferred_element_type=jnp.float32)

    return pl.pallas_call(
        body,
        grid=grid,
        in_specs=[
            pl.BlockSpec((blk, 1), lambda i: (i, 0)),
            pl.BlockSpec((blk, 1), lambda i: (i, 0)),
            pl.BlockSpec((blk, D), lambda i: (i, 0)),
            pl.BlockSpec((D, D), lambda i: (0, 0)),
        ],
        out_specs=[
            pl.BlockSpec((blk, D), lambda i: (i, 0)),
            pl.BlockSpec((blk, 1), lambda i: (i, 0)),
        ],
        out_shape=[
            jax.ShapeDtypeStruct((npad, D), jnp.float32),
            jax.ShapeDtypeStruct((npad, 1), jnp.float32),
        ],
    )(dp0, dp1, x, w1)


def _tc_mid(s0, s1, g1, dinv, b1, w2, npad, blk):
    grid = (npad // blk,)

    def body(s0_ref, s1_ref, g_ref, dinv_ref, b_ref, w_ref, out_ref):
        d = dinv_ref[...]
        t = d * (s0_ref[...] + s1_ref[...] + g_ref[...]) + b_ref[...]
        out_ref[...] = d * jnp.dot(t, w_ref[...],
                                   preferred_element_type=jnp.float32)

    return pl.pallas_call(
        body,
        grid=grid,
        in_specs=[
            pl.BlockSpec((blk, D), lambda i: (i, 0)),
            pl.BlockSpec((blk, D), lambda i: (i, 0)),
            pl.BlockSpec((blk, D), lambda i: (i, 0)),
            pl.BlockSpec((blk, 1), lambda i: (i, 0)),
            pl.BlockSpec((1, D), lambda i: (0, 0)),
            pl.BlockSpec((D, D), lambda i: (0, 0)),
        ],
        out_specs=pl.BlockSpec((blk, D), lambda i: (i, 0)),
        out_shape=jax.ShapeDtypeStruct((npad, D), jnp.float32),
    )(s0, s1, g1, dinv, b1, w2)


def _tc_last(s0, s1, g2, dinv, b2, npad, blk):
    grid = (npad // blk,)

    def body(s0_ref, s1_ref, g_ref, dinv_ref, b_ref, out_ref):
        out_ref[...] = (dinv_ref[...] * (s0_ref[...] + s1_ref[...] + g_ref[...])
                        + b_ref[...])

    return pl.pallas_call(
        body,
        grid=grid,
        in_specs=[
            pl.BlockSpec((blk, D), lambda i: (i, 0)),
            pl.BlockSpec((blk, D), lambda i: (i, 0)),
            pl.BlockSpec((blk, D), lambda i: (i, 0)),
            pl.BlockSpec((blk, 1), lambda i: (i, 0)),
            pl.BlockSpec((1, D), lambda i: (0, 0)),
        ],
        out_specs=pl.BlockSpec((blk, D), lambda i: (i, 0)),
        out_shape=jax.ShapeDtypeStruct((npad, D), jnp.float32),
    )(s0, s1, g2, dinv, b2)


def kernel(x, edge_index, W1, b1, W2, b2, original_size):
    n = x.shape[0]
    e = edge_index.shape[1]

    npad = -(-(n + 1) // 1280) * 1280
    rps = npad // NS

    nch = -(-e // (NC * NS * CH))
    nch += nch % 2
    tot = NC * NS * nch * CH
    src = jnp.concatenate(
        [edge_index[0], jnp.zeros((tot - e,), jnp.int32)])
    dst = jnp.concatenate(
        [edge_index[1], jnp.full((tot - e,), npad - 1, jnp.int32)])
    srcs = src.reshape(NC, NS, nch, CH)
    dsts = dst.reshape(NC, NS, nch, CH)

    x_pad = jnp.pad(x, ((0, npad - n), (0, 0)))
    zdeg = jnp.zeros((rps,), jnp.float32)
    zrows = jnp.zeros((rps, D), jnp.float32)
    b1r = b1.reshape(1, D)
    b2r = b2.reshape(1, D)

    blk = 1280

    dp = _sc_degree(dsts, zdeg, nch, npad, rps)
    dp0 = dp[0].reshape(npad, 1)
    dp1 = dp[1].reshape(npad, 1)

    g1, dinv = _tc_first(dp0, dp1, x_pad, W1, npad, blk)
    s_1 = _sc_aggregate(g1, srcs, dsts, zrows, nch, npad, rps)
    g2 = _tc_mid(s_1[0], s_1[1], g1, dinv, b1r, W2, npad, blk)
    s_2 = _sc_aggregate(g2, srcs, dsts, zrows, nch, npad, rps)
    out = _tc_last(s_2[0], s_2[1], g2, dinv, b2r, npad, blk)
    return out[:n]


# speedup vs baseline: 7.5881x; 7.5881x over previous
import functools

import jax
import jax.numpy as jnp
from jax import lax
from jax.experimental import pallas as pl
from jax.experimental.pallas import tpu as pltpu
from jax.experimental.pallas import tpu_sc as plsc

NC = 2
NS = 16
CH = 128
D = 128
NBLK = 16


def _sc_degree(dsts0, dsts1, zdeg, nch0, nch1, npad, rps):
    mesh = plsc.VectorSubcoreMesh(core_axis_name="c", subcore_axis_name="s")

    @functools.partial(
        pl.kernel,
        out_type=jax.ShapeDtypeStruct((NC * npad,), jnp.float32),
        mesh=mesh,
        scratch_types=[
            pltpu.VMEM((nch0, CH), jnp.int32),
            pltpu.VMEM((CH,), jnp.float32),
            pltpu.VMEM_SHARED((npad,), jnp.float32),
        ],
    )
    def deg_kernel(dsts0_hbm, dsts1_hbm, zdeg_hbm, dp_hbm, idx_v, ones_v, sdeg):
        c = lax.axis_index("c")
        s = lax.axis_index("s")
        for i in range(CH // 16):
            ones_v[pl.ds(i * 16, 16)] = jnp.full((16,), 1.0, jnp.float32)
        pltpu.sync_copy(zdeg_hbm, sdeg.at[pl.ds(s * rps, rps)])

        @pl.when(c == 0)
        def _():
            pltpu.sync_copy(dsts0_hbm.at[s], idx_v.at[pl.ds(0, nch0)])

        @pl.when(c == 1)
        def _():
            pltpu.sync_copy(dsts1_hbm.at[s], idx_v.at[pl.ds(0, nch1)])

        plsc.subcore_barrier()
        nch_c = jnp.where(c == 0, nch0, nch1)

        def body(j, carry):
            pltpu.sync_copy(ones_v, sdeg.at[idx_v.at[j]], add=True)
            return carry

        lax.fori_loop(0, nch_c, body, 0)
        plsc.subcore_barrier()
        pltpu.sync_copy(sdeg.at[pl.ds(s * rps, rps)],
                        dp_hbm.at[pl.ds(c * npad + s * rps, rps)])

    return deg_kernel(dsts0, dsts1, zdeg).reshape(NC, npad)


def _sc_aggregate(g, srcs, dsts, nch, npad, rps):
    nb = nch // NBLK
    mesh = plsc.VectorSubcoreMesh(core_axis_name="c", subcore_axis_name="s")

    @functools.partial(
        pl.kernel,
        out_type=jax.ShapeDtypeStruct((npad, D), jnp.float32),
        mesh=mesh,
        scratch_types=[
            pltpu.VMEM((2, NBLK, CH), jnp.int32),
            pltpu.VMEM((2, NBLK, CH), jnp.int32),
            pltpu.VMEM((2, CH, D), jnp.float32),
            pltpu.VMEM_SHARED((npad, D), jnp.float32),
            pltpu.SemaphoreType.DMA,
            pltpu.SemaphoreType.DMA,
            pltpu.SemaphoreType.DMA,
        ],
    )
    def agg_kernel(g_hbm, srcs_hbm, dsts_hbm, s_hbm,
                   isrc, idst, rows, acc, sem0, sem1, semi):
        c = lax.axis_index("c")
        s = lax.axis_index("s")

        @pl.when(c == 0)
        def _():
            def zrow(i, carry):
                for l in range(D // 16):
                    rows[0, i, pl.ds(l * 16, 16)] = jnp.zeros((16,),
                                                              jnp.float32)
                return carry

            lax.fori_loop(0, CH, zrow, 0)
            for k in range(rps // CH):
                pltpu.sync_copy(rows.at[0],
                                acc.at[pl.ds(s * rps + k * CH, CH)])

            pltpu.async_copy(srcs_hbm.at[s, pl.ds(0, NBLK)], isrc.at[0], semi)
            pltpu.async_copy(dsts_hbm.at[s, pl.ds(0, NBLK)], idst.at[0], semi)
            plsc.subcore_barrier()
            pltpu.make_async_copy(srcs_hbm.at[s, pl.ds(0, NBLK)],
                                  isrc.at[0], semi).wait()
            pltpu.make_async_copy(dsts_hbm.at[s, pl.ds(0, NBLK)],
                                  idst.at[0], semi).wait()

            def bblock(b, carry):
                cur = lax.rem(b, 2)
                nxt = lax.rem(b + 1, 2)

                @pl.when(b + 1 < nb)
                def _():
                    pltpu.async_copy(
                        srcs_hbm.at[s, pl.ds((b + 1) * NBLK, NBLK)],
                        isrc.at[nxt], semi)
                    pltpu.async_copy(
                        dsts_hbm.at[s, pl.ds((b + 1) * NBLK, NBLK)],
                        idst.at[nxt], semi)

                pltpu.async_copy(g_hbm.at[isrc.at[cur, 0]], rows.at[0], sem0)

                def body(k, carry2):
                    j = 2 * k
                    pltpu.async_copy(g_hbm.at[isrc.at[cur, j + 1]],
                                     rows.at[1], sem1)
                    pltpu.make_async_copy(g_hbm.at[pl.ds(0, CH)],
                                          rows.at[0], sem0).wait()
                    pltpu.sync_copy(rows.at[0], acc.at[idst.at[cur, j]],
                                    add=True)

                    @pl.when(k + 1 < NBLK // 2)
                    def _():
                        pltpu.async_copy(g_hbm.at[isrc.at[cur, j + 2]],
                                         rows.at[0], sem0)

                    pltpu.make_async_copy(g_hbm.at[pl.ds(0, CH)],
                                          rows.at[1], sem1).wait()
                    pltpu.sync_copy(rows.at[1], acc.at[idst.at[cur, j + 1]],
                                    add=True)
                    return carry2

                lax.fori_loop(0, NBLK // 2, body, 0)

                @pl.when(b + 1 < nb)
                def _():
                    pltpu.make_async_copy(srcs_hbm.at[s, pl.ds(0, NBLK)],
                                          isrc.at[nxt], semi).wait()
                    pltpu.make_async_copy(dsts_hbm.at[s, pl.ds(0, NBLK)],
                                          idst.at[nxt], semi).wait()
                return carry

            lax.fori_loop(0, nb, bblock, 0)
            plsc.subcore_barrier()
            pltpu.sync_copy(acc.at[pl.ds(s * rps, rps)],
                            s_hbm.at[pl.ds(s * rps, rps)])

    return agg_kernel(g, srcs, dsts)


def _tc_first(dp0, dp1, x, w1, npad, blk):
    grid = (npad // blk,)

    def body(dp0_ref, dp1_ref, x_ref, w_ref, g_ref, dinv_ref):
        d = lax.rsqrt(dp0_ref[...] + dp1_ref[...] + 1.0)
        dinv_ref[...] = d
        g_ref[...] = d * jnp.dot(x_ref[...], w_ref[...],
                                 preferred_element_type=jnp.float32)

    return pl.pallas_call(
        body,
        grid=grid,
        in_specs=[
            pl.BlockSpec((blk, 1), lambda i: (i, 0)),
            pl.BlockSpec((blk, 1), lambda i: (i, 0)),
            pl.BlockSpec((blk, D), lambda i: (i, 0)),
            pl.BlockSpec((D, D), lambda i: (0, 0)),
        ],
        out_specs=[
            pl.BlockSpec((blk, D), lambda i: (i, 0)),
            pl.BlockSpec((blk, 1), lambda i: (i, 0)),
        ],
        out_shape=[
            jax.ShapeDtypeStruct((npad, D), jnp.float32),
            jax.ShapeDtypeStruct((npad, 1), jnp.float32),
        ],
    )(dp0, dp1, x, w1)


def _tc_mid(s1, g1, dinv, b1, w2, npad, blk):
    grid = (npad // blk,)

    def body(s_ref, g_ref, dinv_ref, b_ref, w_ref, out_ref):
        d = dinv_ref[...]
        t = d * (s_ref[...] + g_ref[...]) + b_ref[...]
        out_ref[...] = d * jnp.dot(t, w_ref[...],
                                   preferred_element_type=jnp.float32)

    return pl.pallas_call(
        body,
        grid=grid,
        in_specs=[
            pl.BlockSpec((blk, D), lambda i: (i, 0)),
            pl.BlockSpec((blk, D), lambda i: (i, 0)),
            pl.BlockSpec((blk, 1), lambda i: (i, 0)),
            pl.BlockSpec((1, D), lambda i: (0, 0)),
            pl.BlockSpec((D, D), lambda i: (0, 0)),
        ],
        out_specs=pl.BlockSpec((blk, D), lambda i: (i, 0)),
        out_shape=jax.ShapeDtypeStruct((npad, D), jnp.float32),
    )(s1, g1, dinv, b1, w2)


def _tc_last(s2, g2, dinv, b2, npad, blk):
    grid = (npad // blk,)

    def body(s_ref, g_ref, dinv_ref, b_ref, out_ref):
        out_ref[...] = (dinv_ref[...] * (s_ref[...] + g_ref[...])
                        + b_ref[...])

    return pl.pallas_call(
        body,
        grid=grid,
        in_specs=[
            pl.BlockSpec((blk, D), lambda i: (i, 0)),
            pl.BlockSpec((blk, D), lambda i: (i, 0)),
            pl.BlockSpec((blk, 1), lambda i: (i, 0)),
            pl.BlockSpec((1, D), lambda i: (0, 0)),
        ],
        out_specs=pl.BlockSpec((blk, D), lambda i: (i, 0)),
        out_shape=jax.ShapeDtypeStruct((npad, D), jnp.float32),
    )(s2, g2, dinv, b2)


def kernel(x, edge_index, W1, b1, W2, b2, original_size):
    n = x.shape[0]
    e = edge_index.shape[1]

    npad = -(-(n + 1) // 2048) * 2048
    rps = npad // NS

    unit = NS * NBLK * CH
    units = -(-e // unit)
    tot = units * unit
    nch = units * NBLK
    src = jnp.concatenate(
        [edge_index[0], jnp.zeros((tot - e,), jnp.int32)])
    dst = jnp.concatenate(
        [edge_index[1],
         n + (jnp.arange(tot - e, dtype=jnp.int32) % (npad - n))])
    srcs = src.reshape(NS, nch, CH)
    dsts = dst.reshape(NS, nch, CH)

    u0 = max(1, min(units - 1, (units * 4) // 5)) if units > 1 else units
    u1 = units - u0
    if u1 == 0:
        u0 = units
        u1 = 0
    if u1 == 0:
        dsts0, dsts1 = dsts, dsts[:, :NBLK]
        nch0, nch1 = nch, 0
    else:
        nch0, nch1 = u0 * NBLK, u1 * NBLK
        e0 = NS * nch0 * CH
        dsts0 = dst[:e0].reshape(NS, nch0, CH)
        dsts1 = dst[e0:].reshape(NS, nch1, CH)

    x_pad = jnp.pad(x, ((0, npad - n), (0, 0)))
    zdeg = jnp.zeros((rps,), jnp.float32)
    b1r = b1.reshape(1, D)
    b2r = b2.reshape(1, D)

    blk = npad // 8

    dp = _sc_degree(dsts0, dsts1, zdeg, nch0, nch1, npad, rps)
    dp0 = dp[0].reshape(npad, 1)
    dp1 = dp[1].reshape(npad, 1)

    g1, dinv = _tc_first(dp0, dp1, x_pad, W1, npad, blk)
    s_1 = _sc_aggregate(g1, srcs, dsts, nch, npad, rps)
    g2 = _tc_mid(s_1, g1, dinv, b1r, W2, npad, blk)
    s_2 = _sc_aggregate(g2, srcs, dsts, nch, npad, rps)
    out = _tc_last(s_2, g2, dinv, b2r, npad, blk)
    return out[:n]


# speedup vs baseline: 19.6657x; 2.5916x over previous
import functools

import jax
import jax.numpy as jnp
from jax import lax
from jax.experimental import pallas as pl
from jax.experimental.pallas import tpu as pltpu
from jax.experimental.pallas import tpu_sc as plsc

NC = 2
NS = 16
CH = 128
D = 128
NBLK = 16


def _sc_degree(dsts0, dsts1, zdeg, nch0, nch1, npad, rps):
    mesh = plsc.VectorSubcoreMesh(core_axis_name="c", subcore_axis_name="s")

    @functools.partial(
        pl.kernel,
        out_type=jax.ShapeDtypeStruct((NC * npad,), jnp.float32),
        mesh=mesh,
        scratch_types=[
            pltpu.VMEM((nch0, CH), jnp.int32),
            pltpu.VMEM((CH,), jnp.float32),
            pltpu.VMEM_SHARED((npad,), jnp.float32),
        ],
    )
    def deg_kernel(dsts0_hbm, dsts1_hbm, zdeg_hbm, dp_hbm, idx_v, ones_v, sdeg):
        c = lax.axis_index("c")
        s = lax.axis_index("s")
        for i in range(CH // 16):
            ones_v[pl.ds(i * 16, 16)] = jnp.full((16,), 1.0, jnp.float32)
        pltpu.sync_copy(zdeg_hbm, sdeg.at[pl.ds(s * rps, rps)])

        @pl.when(c == 0)
        def _():
            pltpu.sync_copy(dsts0_hbm.at[s], idx_v.at[pl.ds(0, nch0)])

        @pl.when(c == 1)
        def _():
            pltpu.sync_copy(dsts1_hbm.at[s], idx_v.at[pl.ds(0, nch1)])

        plsc.subcore_barrier()
        nch_c = jnp.where(c == 0, nch0, nch1)

        def body(j, carry):
            pltpu.sync_copy(ones_v, sdeg.at[idx_v.at[j]], add=True)
            return carry

        lax.fori_loop(0, nch_c, body, 0)
        plsc.subcore_barrier()
        pltpu.sync_copy(sdeg.at[pl.ds(s * rps, rps)],
                        dp_hbm.at[pl.ds(c * npad + s * rps, rps)])

    return deg_kernel(dsts0, dsts1, zdeg).reshape(NC, npad)


def _sc_aggregate(g, srcs, dsts, nch, npad, rps):
    nb = nch // NBLK
    mesh = plsc.VectorSubcoreMesh(core_axis_name="c", subcore_axis_name="s")

    @functools.partial(
        pl.kernel,
        out_type=jax.ShapeDtypeStruct((npad, D), jnp.float32),
        mesh=mesh,
        scratch_types=[
            pltpu.VMEM((2, NBLK, CH), jnp.int32),
            pltpu.VMEM((2, NBLK, CH), jnp.int32),
            pltpu.VMEM((2, CH, D), jnp.float32),
            pltpu.VMEM_SHARED((npad, D), jnp.float32),
            pltpu.SemaphoreType.DMA,
            pltpu.SemaphoreType.DMA,
            pltpu.SemaphoreType.DMA,
        ],
    )
    def agg_kernel(g_hbm, srcs_hbm, dsts_hbm, s_hbm,
                   isrc, idst, rows, acc, sem0, sem1, semi):
        c = lax.axis_index("c")
        s = lax.axis_index("s")

        @pl.when(c == 0)
        def _():
            with jax.named_scope("agg_zero"):
                def zrow(i, carry):
                    for l in range(D // 16):
                        rows[0, i, pl.ds(l * 16, 16)] = jnp.zeros((16,),
                                                                  jnp.float32)
                    return carry

                lax.fori_loop(0, CH, zrow, 0)
                for k in range(rps // CH):
                    pltpu.sync_copy(rows.at[0],
                                    acc.at[pl.ds(s * rps + k * CH, CH)])

            pltpu.async_copy(srcs_hbm.at[s, pl.ds(0, NBLK)], isrc.at[0], semi)
            pltpu.async_copy(dsts_hbm.at[s, pl.ds(0, NBLK)], idst.at[0], semi)
            plsc.subcore_barrier()
            pltpu.make_async_copy(srcs_hbm.at[s, pl.ds(0, NBLK)],
                                  isrc.at[0], semi).wait()
            pltpu.make_async_copy(dsts_hbm.at[s, pl.ds(0, NBLK)],
                                  idst.at[0], semi).wait()

            def bblock(b, carry):
                cur = lax.rem(b, 2)
                nxt = lax.rem(b + 1, 2)

                @pl.when(b + 1 < nb)
                def _():
                    pltpu.async_copy(
                        srcs_hbm.at[s, pl.ds((b + 1) * NBLK, NBLK)],
                        isrc.at[nxt], semi)
                    pltpu.async_copy(
                        dsts_hbm.at[s, pl.ds((b + 1) * NBLK, NBLK)],
                        idst.at[nxt], semi)

                pltpu.async_copy(g_hbm.at[isrc.at[cur, 0]], rows.at[0], sem0)

                def body(k, carry2):
                    j = 2 * k
                    pltpu.async_copy(g_hbm.at[isrc.at[cur, j + 1]],
                                     rows.at[1], sem1)
                    pltpu.make_async_copy(g_hbm.at[pl.ds(0, CH)],
                                          rows.at[0], sem0).wait()
                    pltpu.sync_copy(rows.at[0], acc.at[idst.at[cur, j]],
                                    add=True)

                    @pl.when(k + 1 < NBLK // 2)
                    def _():
                        pltpu.async_copy(g_hbm.at[isrc.at[cur, j + 2]],
                                         rows.at[0], sem0)

                    pltpu.make_async_copy(g_hbm.at[pl.ds(0, CH)],
                                          rows.at[1], sem1).wait()
                    pltpu.sync_copy(rows.at[1], acc.at[idst.at[cur, j + 1]],
                                    add=True)
                    return carry2

                lax.fori_loop(0, NBLK // 2, body, 0)

                @pl.when(b + 1 < nb)
                def _():
                    pltpu.make_async_copy(srcs_hbm.at[s, pl.ds(0, NBLK)],
                                          isrc.at[nxt], semi).wait()
                    pltpu.make_async_copy(dsts_hbm.at[s, pl.ds(0, NBLK)],
                                          idst.at[nxt], semi).wait()
                return carry

            with jax.named_scope("agg_edges"):
                lax.fori_loop(0, nb, bblock, 0)
            plsc.subcore_barrier()
            with jax.named_scope("agg_out"):
                pltpu.sync_copy(acc.at[pl.ds(s * rps, rps)],
                                s_hbm.at[pl.ds(s * rps, rps)])

    return agg_kernel(g, srcs, dsts)


def _tc_first(dp0, dp1, x, w1, npad, blk):
    grid = (npad // blk,)

    def body(dp0_ref, dp1_ref, x_ref, w_ref, g_ref, dinv_ref):
        d = lax.rsqrt(dp0_ref[...] + dp1_ref[...] + 1.0)
        dinv_ref[...] = d
        g_ref[...] = d * jnp.dot(x_ref[...], w_ref[...],
                                 preferred_element_type=jnp.float32)

    return pl.pallas_call(
        body,
        grid=grid,
        in_specs=[
            pl.BlockSpec((blk, 1), lambda i: (i, 0)),
            pl.BlockSpec((blk, 1), lambda i: (i, 0)),
            pl.BlockSpec((blk, D), lambda i: (i, 0)),
            pl.BlockSpec((D, D), lambda i: (0, 0)),
        ],
        out_specs=[
            pl.BlockSpec((blk, D), lambda i: (i, 0)),
            pl.BlockSpec((blk, 1), lambda i: (i, 0)),
        ],
        out_shape=[
            jax.ShapeDtypeStruct((npad, D), jnp.float32),
            jax.ShapeDtypeStruct((npad, 1), jnp.float32),
        ],
    )(dp0, dp1, x, w1)


def _tc_mid(s1, g1, dinv, b1, w2, npad, blk):
    grid = (npad // blk,)

    def body(s_ref, g_ref, dinv_ref, b_ref, w_ref, out_ref):
        d = dinv_ref[...]
        t = d * (s_ref[...] + g_ref[...]) + b_ref[...]
        out_ref[...] = d * jnp.dot(t, w_ref[...],
                                   preferred_element_type=jnp.float32)

    return pl.pallas_call(
        body,
        grid=grid,
        in_specs=[
            pl.BlockSpec((blk, D), lambda i: (i, 0)),
            pl.BlockSpec((blk, D), lambda i: (i, 0)),
            pl.BlockSpec((blk, 1), lambda i: (i, 0)),
            pl.BlockSpec((1, D), lambda i: (0, 0)),
            pl.BlockSpec((D, D), lambda i: (0, 0)),
        ],
        out_specs=pl.BlockSpec((blk, D), lambda i: (i, 0)),
        out_shape=jax.ShapeDtypeStruct((npad, D), jnp.float32),
    )(s1, g1, dinv, b1, w2)


def _tc_last(s2, g2, dinv, b2, npad, blk):
    grid = (npad // blk,)

    def body(s_ref, g_ref, dinv_ref, b_ref, out_ref):
        out_ref[...] = (dinv_ref[...] * (s_ref[...] + g_ref[...])
                        + b_ref[...])

    return pl.pallas_call(
        body,
        grid=grid,
        in_specs=[
            pl.BlockSpec((blk, D), lambda i: (i, 0)),
            pl.BlockSpec((blk, D), lambda i: (i, 0)),
            pl.BlockSpec((blk, 1), lambda i: (i, 0)),
            pl.BlockSpec((1, D), lambda i: (0, 0)),
        ],
        out_specs=pl.BlockSpec((blk, D), lambda i: (i, 0)),
        out_shape=jax.ShapeDtypeStruct((npad, D), jnp.float32),
    )(s2, g2, dinv, b2)


def kernel(x, edge_index, W1, b1, W2, b2, original_size):
    n = x.shape[0]
    e = edge_index.shape[1]

    npad = -(-(n + 1) // 2048) * 2048
    rps = npad // NS

    unit = NS * NBLK * CH
    units = -(-e // unit)
    tot = units * unit
    nch = units * NBLK
    src = jnp.concatenate(
        [edge_index[0], jnp.arange(tot - e, dtype=jnp.int32) % n])
    dst = jnp.concatenate(
        [edge_index[1],
         n + (jnp.arange(tot - e, dtype=jnp.int32) % (npad - n))])
    srcs = src.reshape(nch, NS, CH).swapaxes(0, 1)
    dsts = dst.reshape(nch, NS, CH).swapaxes(0, 1)

    u0 = max(1, min(units - 1, (units * 4) // 5)) if units > 1 else units
    u1 = units - u0
    if u1 == 0:
        u0 = units
        u1 = 0
    if u1 == 0:
        dsts0, dsts1 = dsts, dsts[:, :NBLK]
        nch0, nch1 = nch, 0
    else:
        nch0, nch1 = u0 * NBLK, u1 * NBLK
        e0 = NS * nch0 * CH
        dsts0 = dst[:e0].reshape(NS, nch0, CH)
        dsts1 = dst[e0:].reshape(NS, nch1, CH)

    x_pad = jnp.pad(x, ((0, npad - n), (0, 0)))
    zdeg = jnp.zeros((rps,), jnp.float32)
    b1r = b1.reshape(1, D)
    b2r = b2.reshape(1, D)

    blk = npad // 8

    dp = _sc_degree(dsts0, dsts1, zdeg, nch0, nch1, npad, rps)
    dp0 = dp[0].reshape(npad, 1)
    dp1 = dp[1].reshape(npad, 1)

    g1, dinv = _tc_first(dp0, dp1, x_pad, W1, npad, blk)
    s_1 = _sc_aggregate(g1, srcs, dsts, nch, npad, rps)
    g2 = _tc_mid(s_1, g1, dinv, b1r, W2, npad, blk)
    s_2 = _sc_aggregate(g2, srcs, dsts, nch, npad, rps)
    out = _tc_last(s_2, g2, dinv, b2r, npad, blk)
    return out[:n]


# speedup vs baseline: 30.2973x; 1.5406x over previous
import functools

import jax
import jax.numpy as jnp
from jax import lax
from jax.experimental import pallas as pl
from jax.experimental.pallas import tpu as pltpu
from jax.experimental.pallas import tpu_sc as plsc

NC = 2
NS = 16
CH = 128
D = 128
NBLK = 16


def _sc_degree(dsts, zdeg, nch, npad, rps):
    mesh = plsc.VectorSubcoreMesh(core_axis_name="c", subcore_axis_name="s")

    @functools.partial(
        pl.kernel,
        out_type=jax.ShapeDtypeStruct((NC * npad,), jnp.float32),
        mesh=mesh,
        scratch_types=[
            pltpu.VMEM((nch, CH), jnp.int32),
            pltpu.VMEM((CH,), jnp.float32),
            pltpu.VMEM_SHARED((npad,), jnp.float32),
        ],
    )
    def deg_kernel(dsts_hbm, zdeg_hbm, dp_hbm, idx_v, ones_v, sdeg):
        c = lax.axis_index("c")
        s = lax.axis_index("s")
        for i in range(CH // 16):
            ones_v[pl.ds(i * 16, 16)] = jnp.full((16,), 1.0, jnp.float32)
        pltpu.sync_copy(zdeg_hbm, sdeg.at[pl.ds(s * rps, rps)])
        pltpu.sync_copy(dsts_hbm.at[c, s], idx_v)
        plsc.subcore_barrier()

        def body(j, carry):
            pltpu.sync_copy(ones_v, sdeg.at[idx_v.at[j]], add=True)
            return carry

        lax.fori_loop(0, nch, body, 0)
        plsc.subcore_barrier()
        pltpu.sync_copy(sdeg.at[pl.ds(s * rps, rps)],
                        dp_hbm.at[pl.ds(c * npad + s * rps, rps)])

    return deg_kernel(dsts, zdeg).reshape(NC, npad)


def _sc_aggregate(g, srcs, dsts, nch, npad, rps):
    nb = nch // NBLK
    mesh = plsc.VectorSubcoreMesh(core_axis_name="c", subcore_axis_name="s")

    @functools.partial(
        pl.kernel,
        out_type=jax.ShapeDtypeStruct((NC, npad, D), jnp.float32),
        mesh=mesh,
        scratch_types=[
            pltpu.VMEM((2, NBLK, CH), jnp.int32),
            pltpu.VMEM((2, NBLK, CH), jnp.int32),
            pltpu.VMEM((2, CH, D), jnp.float32),
            pltpu.VMEM_SHARED((npad, D), jnp.float32),
            pltpu.SemaphoreType.DMA,
            pltpu.SemaphoreType.DMA,
            pltpu.SemaphoreType.DMA,
        ],
    )
    def agg_kernel(g_hbm, srcs_hbm, dsts_hbm, s_hbm,
                   isrc, idst, rows, acc, sem0, sem1, semi):
        c = lax.axis_index("c")
        s = lax.axis_index("s")

        def zrow(i, carry):
            for l in range(D // 16):
                rows[0, i, pl.ds(l * 16, 16)] = jnp.zeros((16,),
                                                          jnp.float32)
            return carry

        lax.fori_loop(0, CH, zrow, 0)
        for k in range(rps // CH):
            pltpu.sync_copy(rows.at[0],
                            acc.at[pl.ds(s * rps + k * CH, CH)])

        pltpu.async_copy(srcs_hbm.at[c, s, pl.ds(0, NBLK)], isrc.at[0], semi)
        pltpu.async_copy(dsts_hbm.at[c, s, pl.ds(0, NBLK)], idst.at[0], semi)
        plsc.subcore_barrier()
        pltpu.make_async_copy(srcs_hbm.at[c, s, pl.ds(0, NBLK)],
                              isrc.at[0], semi).wait()
        pltpu.make_async_copy(dsts_hbm.at[c, s, pl.ds(0, NBLK)],
                              idst.at[0], semi).wait()

        def bblock(b, carry):
            cur = lax.rem(b, 2)
            nxt = lax.rem(b + 1, 2)

            @pl.when(b + 1 < nb)
            def _():
                pltpu.async_copy(
                    srcs_hbm.at[c, s, pl.ds((b + 1) * NBLK, NBLK)],
                    isrc.at[nxt], semi)
                pltpu.async_copy(
                    dsts_hbm.at[c, s, pl.ds((b + 1) * NBLK, NBLK)],
                    idst.at[nxt], semi)

            pltpu.async_copy(g_hbm.at[isrc.at[cur, 0]], rows.at[0], sem0)

            def body(k, carry2):
                j = 2 * k
                pltpu.async_copy(g_hbm.at[isrc.at[cur, j + 1]],
                                 rows.at[1], sem1)
                pltpu.make_async_copy(g_hbm.at[pl.ds(0, CH)],
                                      rows.at[0], sem0).wait()
                pltpu.sync_copy(rows.at[0], acc.at[idst.at[cur, j]],
                                add=True)

                @pl.when(k + 1 < NBLK // 2)
                def _():
                    pltpu.async_copy(g_hbm.at[isrc.at[cur, j + 2]],
                                     rows.at[0], sem0)

                pltpu.make_async_copy(g_hbm.at[pl.ds(0, CH)],
                                      rows.at[1], sem1).wait()
                pltpu.sync_copy(rows.at[1], acc.at[idst.at[cur, j + 1]],
                                add=True)
                return carry2

            lax.fori_loop(0, NBLK // 2, body, 0)

            @pl.when(b + 1 < nb)
            def _():
                pltpu.make_async_copy(srcs_hbm.at[c, s, pl.ds(0, NBLK)],
                                      isrc.at[nxt], semi).wait()
                pltpu.make_async_copy(dsts_hbm.at[c, s, pl.ds(0, NBLK)],
                                      idst.at[nxt], semi).wait()
            return carry

        lax.fori_loop(0, nb, bblock, 0)
        plsc.subcore_barrier()
        pltpu.sync_copy(acc.at[pl.ds(s * rps, rps)],
                        s_hbm.at[c, pl.ds(s * rps, rps)])

    return agg_kernel(g, srcs, dsts)


def _tc_first(dp0, dp1, x, w1, npad, blk):
    grid = (npad // blk,)

    def body(dp0_ref, dp1_ref, x_ref, w_ref, g_ref, dinv_ref):
        d = lax.rsqrt(dp0_ref[...] + dp1_ref[...] + 1.0)
        dinv_ref[...] = d
        g_ref[...] = d * jnp.dot(x_ref[...], w_ref[...],
                                 preferred_element_type=jnp.float32)

    return pl.pallas_call(
        body,
        grid=grid,
        in_specs=[
            pl.BlockSpec((blk, 1), lambda i: (i, 0)),
            pl.BlockSpec((blk, 1), lambda i: (i, 0)),
            pl.BlockSpec((blk, D), lambda i: (i, 0)),
            pl.BlockSpec((D, D), lambda i: (0, 0)),
        ],
        out_specs=[
            pl.BlockSpec((blk, D), lambda i: (i, 0)),
            pl.BlockSpec((blk, 1), lambda i: (i, 0)),
        ],
        out_shape=[
            jax.ShapeDtypeStruct((npad, D), jnp.float32),
            jax.ShapeDtypeStruct((npad, 1), jnp.float32),
        ],
    )(dp0, dp1, x, w1)


def _tc_mid(s0, s1, g1, dinv, b1, w2, npad, blk):
    grid = (npad // blk,)

    def body(s0_ref, s1_ref, g_ref, dinv_ref, b_ref, w_ref, out_ref):
        d = dinv_ref[...]
        t = d * (s0_ref[...] + s1_ref[...] + g_ref[...]) + b_ref[...]
        out_ref[...] = d * jnp.dot(t, w_ref[...],
                                   preferred_element_type=jnp.float32)

    return pl.pallas_call(
        body,
        grid=grid,
        in_specs=[
            pl.BlockSpec((blk, D), lambda i: (i, 0)),
            pl.BlockSpec((blk, D), lambda i: (i, 0)),
            pl.BlockSpec((blk, D), lambda i: (i, 0)),
            pl.BlockSpec((blk, 1), lambda i: (i, 0)),
            pl.BlockSpec((1, D), lambda i: (0, 0)),
            pl.BlockSpec((D, D), lambda i: (0, 0)),
        ],
        out_specs=pl.BlockSpec((blk, D), lambda i: (i, 0)),
        out_shape=jax.ShapeDtypeStruct((npad, D), jnp.float32),
    )(s0, s1, g1, dinv, b1, w2)


def _tc_last(s0, s1, g2, dinv, b2, npad, blk):
    grid = (npad // blk,)

    def body(s0_ref, s1_ref, g_ref, dinv_ref, b_ref, out_ref):
        out_ref[...] = (dinv_ref[...]
                        * (s0_ref[...] + s1_ref[...] + g_ref[...])
                        + b_ref[...])

    return pl.pallas_call(
        body,
        grid=grid,
        in_specs=[
            pl.BlockSpec((blk, D), lambda i: (i, 0)),
            pl.BlockSpec((blk, D), lambda i: (i, 0)),
            pl.BlockSpec((blk, D), lambda i: (i, 0)),
            pl.BlockSpec((blk, 1), lambda i: (i, 0)),
            pl.BlockSpec((1, D), lambda i: (0, 0)),
        ],
        out_specs=pl.BlockSpec((blk, D), lambda i: (i, 0)),
        out_shape=jax.ShapeDtypeStruct((npad, D), jnp.float32),
    )(s0, s1, g2, dinv, b2)


def kernel(x, edge_index, W1, b1, W2, b2, original_size):
    n = x.shape[0]
    e = edge_index.shape[1]

    npad = -(-(n + 1) // 2048) * 2048
    rps = npad // NS

    unit = NC * NS * NBLK * CH
    units = -(-e // unit)
    tot = units * unit
    nch = units * NBLK
    src = jnp.concatenate(
        [edge_index[0], jnp.arange(tot - e, dtype=jnp.int32) % n])
    dst = jnp.concatenate(
        [edge_index[1],
         n + (jnp.arange(tot - e, dtype=jnp.int32) % (npad - n))])
    srcs = src.reshape(nch, NC, NS, CH).transpose(1, 2, 0, 3)
    dsts = dst.reshape(nch, NC, NS, CH).transpose(1, 2, 0, 3)

    x_pad = jnp.pad(x, ((0, npad - n), (0, 0)))
    zdeg = jnp.zeros((rps,), jnp.float32)
    b1r = b1.reshape(1, D)
    b2r = b2.reshape(1, D)

    blk = npad // 8

    dp = _sc_degree(dsts, zdeg, nch, npad, rps)
    dp0 = dp[0].reshape(npad, 1)
    dp1 = dp[1].reshape(npad, 1)

    g1, dinv = _tc_first(dp0, dp1, x_pad, W1, npad, blk)
    s_1 = _sc_aggregate(g1, srcs, dsts, nch, npad, rps)
    g2 = _tc_mid(s_1[0], s_1[1], g1, dinv, b1r, W2, npad, blk)
    s_2 = _sc_aggregate(g2, srcs, dsts, nch, npad, rps)
    out = _tc_last(s_2[0], s_2[1], g2, dinv, b2r, npad, blk)
    return out[:n]
